# baseline (speedup 1.0000x reference)
import dataclasses
import functools

import jax
import jax.numpy as jnp
from jax import lax
from jax.experimental import pallas as pl
from jax.experimental.pallas import tpu as pltpu
from jax.experimental.pallas import tpu_sc as plsc

NC = 2
NS = 16
NW = NC * NS
LANES = 16
CHUNK = 128


def _compiler_params():
    return pltpu.CompilerParams(use_tc_tiling_on_sc=False,
                                needs_layout_passes=False)


@functools.partial(jax.jit, static_argnames=("B", "C", "E"))
def _anime2vec_sc(target, ctx_flat, target_table, context_table, *, B, C, E):
    BPW = B // NW
    RPW = BPW * C
    SUPER = 16
    GROUP = SUPER * C
    mesh = plsc.VectorSubcoreMesh(core_axis_name="c", subcore_axis_name="s")

    @functools.partial(
        pl.kernel,
        mesh=mesh,
        out_type=jax.ShapeDtypeStruct((B * C,), jnp.float32),
        scratch_types=[
            pltpu.VMEM((BPW,), jnp.int32),
            pltpu.VMEM((RPW,), jnp.int32),
            pltpu.VMEM((BPW, E), jnp.float32),
            pltpu.VMEM((RPW, E), jnp.float32),
            pltpu.VMEM((GROUP, LANES), jnp.float32),
            pltpu.VMEM((RPW,), jnp.float32),
            pltpu.SemaphoreType.DMA,
            pltpu.SemaphoreType.DMA,
        ],
        compiler_params=_compiler_params(),
    )
    def k(tgt_hbm, ctx_hbm, ttab_hbm, ctab_hbm, out_hbm,
          tgt_idx, ctx_idx, tgt_rows, ctx_rows, scr, out_v, sem_t, sem_c):
        wid = lax.axis_index("s") * NC + lax.axis_index("c")
        b0 = wid * BPW
        r0 = wid * RPW

        pltpu.sync_copy(tgt_hbm.at[pl.ds(b0, BPW)], tgt_idx)
        pltpu.sync_copy(ctx_hbm.at[pl.ds(r0, RPW)], ctx_idx)

        copies = []
        for j in range(BPW // CHUNK):
            copies.append(pltpu.async_copy(
                ttab_hbm.at[tgt_idx.at[pl.ds(j * CHUNK, CHUNK)]],
                tgt_rows.at[pl.ds(j * CHUNK, CHUNK)], sem_t))
        for j in range(RPW // CHUNK):
            copies.append(pltpu.async_copy(
                ctab_hbm.at[ctx_idx.at[pl.ds(j * CHUNK, CHUNK)]],
                ctx_rows.at[pl.ds(j * CHUNK, CHUNK)], sem_c))
        for cp in copies:
            cp.wait()

        iota16 = lax.iota(jnp.int32, 16)

        @pl.loop(0, BPW // SUPER)
        def _(g):
            base_b = g * SUPER
            base_r = g * GROUP
            for jb in range(SUPER):
                b = base_b + jb
                t0 = tgt_rows[b, pl.ds(0, LANES)]
                t1 = tgt_rows[b, pl.ds(LANES, LANES)]
                for c in range(C):
                    rr = jb * C + c
                    r = base_r + rr
                    p = (ctx_rows[r, pl.ds(0, LANES)] * t0
                         + ctx_rows[r, pl.ds(LANES, LANES)] * t1)
                    scr[rr, :] = p
            for blk in range(GROUP // LANES):
                rows = blk * LANES + iota16
                acc = plsc.load_gather(
                    scr, [rows, jnp.zeros((LANES,), jnp.int32)])
                for e in range(1, LANES):
                    acc = acc + plsc.load_gather(
                        scr, [rows, jnp.full((LANES,), e, jnp.int32)])
                out_v[pl.ds(base_r + blk * LANES, LANES)] = acc

        pltpu.sync_copy(out_v, out_hbm.at[pl.ds(r0, RPW)])

    return k(target, ctx_flat, target_table, context_table)


def kernel(target, context, target_table, context_table):
    B, = target.shape
    _, C = context.shape
    _, E = target_table.shape
    out_flat = _anime2vec_sc(
        target, context.reshape(B * C), target_table, context_table,
        B=B, C=C, E=E)
    return out_flat.reshape(B, C)
